# initial kernel scaffold (unmeasured)
import jax
import jax.numpy as jnp
from jax import lax
from jax.experimental import pallas as pl
from jax.experimental.pallas import tpu as pltpu

T = 2048
T_SHARD = 1024
D = 1024
F = 4096
E = 16
E_LOCAL = 8

T_TILE = 512
F_TILE = 1024


def _gather_gate(x_shard, router_shard):

    def body(x_ref, r_ref, xfull_ref, wsel_ref, recv_x, recv_r, sx, rx, sr, rr):
        my_x = lax.axis_index("x")
        my_y = lax.axis_index("y")
        nbr = (my_x, 1 - my_y)

        barrier = pltpu.get_barrier_semaphore()
        pl.semaphore_signal(
            barrier, inc=1, device_id=nbr, device_id_type=pl.DeviceIdType.MESH
        )
        pl.semaphore_wait(barrier, 1)

        rdma_x = pltpu.make_async_remote_copy(
            src_ref=x_ref, dst_ref=recv_x, send_sem=sx, recv_sem=rx,
            device_id=nbr, device_id_type=pl.DeviceIdType.MESH,
        )
        rdma_x.start()
        rdma_r = pltpu.make_async_remote_copy(
            src_ref=r_ref, dst_ref=recv_r, send_sem=sr, recv_sem=rr,
            device_id=nbr, device_id_type=pl.DeviceIdType.MESH,
        )
        rdma_r.start()
        rdma_x.wait()
        rdma_r.wait()

        xfull_ref[pl.ds(my_y * T_SHARD, T_SHARD), :] = x_ref[...]
        xfull_ref[pl.ds((1 - my_y) * T_SHARD, T_SHARD), :] = recv_x[...]

        xfull = xfull_ref[...]
        g_local = jnp.dot(xfull, r_ref[...], preferred_element_type=jnp.float32)
        g_remote = jnp.dot(xfull, recv_r[...], preferred_element_type=jnp.float32)
        is0 = my_y == 0
        gates = jnp.where(
            is0,
            jnp.concatenate([g_local, g_remote], axis=1),
            jnp.concatenate([g_remote, g_local], axis=1),
        )

        idx = lax.broadcasted_iota(jnp.int32, (T, E), 1)
        m1 = jnp.max(gates, axis=1, keepdims=True)
        i1 = jnp.min(jnp.where(gates == m1, idx, E), axis=1, keepdims=True)
        top1 = idx == i1
        g2 = jnp.where(top1, -jnp.inf, gates)
        m2 = jnp.max(g2, axis=1, keepdims=True)
        i2 = jnp.min(jnp.where(g2 == m2, idx, E), axis=1, keepdims=True)
        top2 = idx == i2
        z = jnp.exp(m2 - m1)
        w1 = 1.0 / (1.0 + z)
        w2 = z / (1.0 + z)
        wdense = jnp.where(top1, w1, 0.0) + jnp.where(top2, w2, 0.0)
        wsel_ref[...] = jnp.where(is0, wdense[:, :E_LOCAL], wdense[:, E_LOCAL:])

    return pl.pallas_call(
        body,
        out_shape=[
            jax.ShapeDtypeStruct((T, D), jnp.float32),
            jax.ShapeDtypeStruct((T, E_LOCAL), jnp.float32),
        ],
        in_specs=[
            pl.BlockSpec(memory_space=pltpu.VMEM),
            pl.BlockSpec(memory_space=pltpu.VMEM),
        ],
        out_specs=[
            pl.BlockSpec(memory_space=pltpu.VMEM),
            pl.BlockSpec(memory_space=pltpu.VMEM),
        ],
        scratch_shapes=[
            pltpu.VMEM((T_SHARD, D), jnp.float32),
            pltpu.VMEM((D, E_LOCAL), jnp.float32),
            pltpu.SemaphoreType.DMA,
            pltpu.SemaphoreType.DMA,
            pltpu.SemaphoreType.DMA,
            pltpu.SemaphoreType.DMA,
        ],
        compiler_params=pltpu.CompilerParams(collective_id=0),
    )(x_shard, router_shard)


def _experts(x_full, w_sel, W1, W2):
    n_t = T // T_TILE
    n_f = F // F_TILE

    def body(x_ref, w_ref, w1_ref, w2_ref, out_ref):
        e = pl.program_id(1)
        f = pl.program_id(2)

        @pl.when((e == 0) & (f == 0))
        def _():
            out_ref[...] = jnp.zeros_like(out_ref)

        h = jnp.maximum(
            jnp.dot(x_ref[...], w1_ref[0], preferred_element_type=jnp.float32),
            0.0,
        )
        col = jnp.sum(
            jnp.where(
                lax.broadcasted_iota(jnp.int32, (T_TILE, E_LOCAL), 1) == e,
                w_ref[...],
                0.0,
            ),
            axis=1,
            keepdims=True,
        )
        out_ref[...] += jnp.dot(
            h * col, w2_ref[0], preferred_element_type=jnp.float32
        )

    return pl.pallas_call(
        body,
        grid=(n_t, E_LOCAL, n_f),
        in_specs=[
            pl.BlockSpec((T_TILE, D), lambda t, e, f: (t, 0)),
            pl.BlockSpec((T_TILE, E_LOCAL), lambda t, e, f: (t, 0)),
            pl.BlockSpec((1, D, F_TILE), lambda t, e, f: (e, 0, f)),
            pl.BlockSpec((1, F_TILE, D), lambda t, e, f: (e, f, 0)),
        ],
        out_specs=pl.BlockSpec((T_TILE, D), lambda t, e, f: (t, 0)),
        out_shape=jax.ShapeDtypeStruct((T, D), jnp.float32),
        compiler_params=pltpu.CompilerParams(
            dimension_semantics=("arbitrary", "arbitrary", "arbitrary"),
        ),
    )(x_full, w_sel, W1, W2)


def _reduce_scatter(partial):

    def body(p_ref, out_ref, send_buf, recv_buf, ss, rs):
        my_x = lax.axis_index("x")
        my_y = lax.axis_index("y")
        nbr = (my_x, 1 - my_y)

        barrier = pltpu.get_barrier_semaphore()
        pl.semaphore_signal(
            barrier, inc=1, device_id=nbr, device_id_type=pl.DeviceIdType.MESH
        )
        pl.semaphore_wait(barrier, 1)

        send_buf[...] = p_ref[pl.ds((1 - my_y) * T_SHARD, T_SHARD), :]
        rdma = pltpu.make_async_remote_copy(
            src_ref=send_buf, dst_ref=recv_buf, send_sem=ss, recv_sem=rs,
            device_id=nbr, device_id_type=pl.DeviceIdType.MESH,
        )
        rdma.start()
        rdma.wait()
        out_ref[...] = p_ref[pl.ds(my_y * T_SHARD, T_SHARD), :] + recv_buf[...]

    return pl.pallas_call(
        body,
        out_shape=jax.ShapeDtypeStruct((T_SHARD, D), jnp.float32),
        in_specs=[pl.BlockSpec(memory_space=pltpu.VMEM)],
        out_specs=pl.BlockSpec(memory_space=pltpu.VMEM),
        scratch_shapes=[
            pltpu.VMEM((T_SHARD, D), jnp.float32),
            pltpu.VMEM((T_SHARD, D), jnp.float32),
            pltpu.SemaphoreType.DMA,
            pltpu.SemaphoreType.DMA,
        ],
        compiler_params=pltpu.CompilerParams(collective_id=1),
    )(partial)


def kernel(x, router, W1, W2):
    x_full, w_sel = _gather_gate(x, router)
    partial = _experts(x_full, w_sel, W1, W2)
    return _reduce_scatter(partial)


# baseline (device time: 516478 ns/iter reference)
import jax
import jax.numpy as jnp
from jax import lax
from jax.experimental import pallas as pl
from jax.experimental.pallas import tpu as pltpu

T = 2048
T_SHARD = 1024
D = 1024
F = 4096
E = 16
E_LOCAL = 8

T_TILE = 512
F_TILE = 1024


def _gather_gate(x_shard, router_shard):

    def body(x_ref, r_ref, xfull_ref, wsel_ref, recv_x, recv_r, sx, rx, sr, rr):
        my_x = lax.axis_index("x")
        my_y = lax.axis_index("y")
        nbr = (my_x, 1 - my_y)

        barrier = pltpu.get_barrier_semaphore()
        pl.semaphore_signal(
            barrier, inc=1, device_id=nbr, device_id_type=pl.DeviceIdType.MESH
        )
        pl.semaphore_wait(barrier, 1)

        rdma_x = pltpu.make_async_remote_copy(
            src_ref=x_ref, dst_ref=recv_x, send_sem=sx, recv_sem=rx,
            device_id=nbr, device_id_type=pl.DeviceIdType.MESH,
        )
        rdma_x.start()
        rdma_r = pltpu.make_async_remote_copy(
            src_ref=r_ref, dst_ref=recv_r, send_sem=sr, recv_sem=rr,
            device_id=nbr, device_id_type=pl.DeviceIdType.MESH,
        )
        rdma_r.start()
        rdma_x.wait()
        rdma_r.wait()

        xfull_ref[pl.ds(my_y * T_SHARD, T_SHARD), :] = x_ref[...]
        xfull_ref[pl.ds((1 - my_y) * T_SHARD, T_SHARD), :] = recv_x[...]

        xfull = xfull_ref[...]
        g_local = jnp.dot(
            xfull, r_ref[...],
            preferred_element_type=jnp.float32,
            precision=lax.Precision.HIGHEST,
        )
        g_remote = jnp.dot(
            xfull, recv_r[...],
            preferred_element_type=jnp.float32,
            precision=lax.Precision.HIGHEST,
        )
        is0 = my_y == 0
        gates = jnp.where(
            is0,
            jnp.concatenate([g_local, g_remote], axis=1),
            jnp.concatenate([g_remote, g_local], axis=1),
        )

        idx = lax.broadcasted_iota(jnp.int32, (T, E), 1)
        m1 = jnp.max(gates, axis=1, keepdims=True)
        i1 = jnp.min(jnp.where(gates == m1, idx, E), axis=1, keepdims=True)
        top1 = idx == i1
        g2 = jnp.where(top1, -jnp.inf, gates)
        m2 = jnp.max(g2, axis=1, keepdims=True)
        i2 = jnp.min(jnp.where(g2 == m2, idx, E), axis=1, keepdims=True)
        top2 = idx == i2
        z = jnp.exp(m2 - m1)
        w1 = 1.0 / (1.0 + z)
        w2 = z / (1.0 + z)
        wdense = jnp.where(top1, w1, 0.0) + jnp.where(top2, w2, 0.0)
        wsel_ref[...] = jnp.where(is0, wdense[:, :E_LOCAL], wdense[:, E_LOCAL:])

    return pl.pallas_call(
        body,
        out_shape=[
            jax.ShapeDtypeStruct((T, D), jnp.float32),
            jax.ShapeDtypeStruct((T, E_LOCAL), jnp.float32),
        ],
        in_specs=[
            pl.BlockSpec(memory_space=pltpu.VMEM),
            pl.BlockSpec(memory_space=pltpu.VMEM),
        ],
        out_specs=[
            pl.BlockSpec(memory_space=pltpu.VMEM),
            pl.BlockSpec(memory_space=pltpu.VMEM),
        ],
        scratch_shapes=[
            pltpu.VMEM((T_SHARD, D), jnp.float32),
            pltpu.VMEM((D, E_LOCAL), jnp.float32),
            pltpu.SemaphoreType.DMA,
            pltpu.SemaphoreType.DMA,
            pltpu.SemaphoreType.DMA,
            pltpu.SemaphoreType.DMA,
        ],
        compiler_params=pltpu.CompilerParams(collective_id=0),
    )(x_shard, router_shard)


def _experts(x_full, w_sel, W1, W2):
    n_t = T // T_TILE
    n_f = F // F_TILE

    def body(x_ref, w_ref, w1_ref, w2_ref, out_ref):
        e = pl.program_id(1)
        f = pl.program_id(2)

        @pl.when((e == 0) & (f == 0))
        def _():
            out_ref[...] = jnp.zeros_like(out_ref)

        h = jnp.maximum(
            jnp.dot(x_ref[...], w1_ref[0], preferred_element_type=jnp.float32),
            0.0,
        )
        col = jnp.sum(
            jnp.where(
                lax.broadcasted_iota(jnp.int32, (T_TILE, E_LOCAL), 1) == e,
                w_ref[...],
                0.0,
            ),
            axis=1,
            keepdims=True,
        )
        out_ref[...] += jnp.dot(
            h * col, w2_ref[0], preferred_element_type=jnp.float32
        )

    return pl.pallas_call(
        body,
        grid=(n_t, E_LOCAL, n_f),
        in_specs=[
            pl.BlockSpec((T_TILE, D), lambda t, e, f: (t, 0)),
            pl.BlockSpec((T_TILE, E_LOCAL), lambda t, e, f: (t, 0)),
            pl.BlockSpec((1, D, F_TILE), lambda t, e, f: (e, 0, f)),
            pl.BlockSpec((1, F_TILE, D), lambda t, e, f: (e, f, 0)),
        ],
        out_specs=pl.BlockSpec((T_TILE, D), lambda t, e, f: (t, 0)),
        out_shape=jax.ShapeDtypeStruct((T, D), jnp.float32),
        compiler_params=pltpu.CompilerParams(
            dimension_semantics=("arbitrary", "arbitrary", "arbitrary"),
        ),
    )(x_full, w_sel, W1, W2)


def _reduce_scatter(partial):

    def body(p_ref, out_ref, send_buf, recv_buf, ss, rs):
        my_x = lax.axis_index("x")
        my_y = lax.axis_index("y")
        nbr = (my_x, 1 - my_y)

        barrier = pltpu.get_barrier_semaphore()
        pl.semaphore_signal(
            barrier, inc=1, device_id=nbr, device_id_type=pl.DeviceIdType.MESH
        )
        pl.semaphore_wait(barrier, 1)

        send_buf[...] = p_ref[pl.ds((1 - my_y) * T_SHARD, T_SHARD), :]
        rdma = pltpu.make_async_remote_copy(
            src_ref=send_buf, dst_ref=recv_buf, send_sem=ss, recv_sem=rs,
            device_id=nbr, device_id_type=pl.DeviceIdType.MESH,
        )
        rdma.start()
        rdma.wait()
        out_ref[...] = p_ref[pl.ds(my_y * T_SHARD, T_SHARD), :] + recv_buf[...]

    return pl.pallas_call(
        body,
        out_shape=jax.ShapeDtypeStruct((T_SHARD, D), jnp.float32),
        in_specs=[pl.BlockSpec(memory_space=pltpu.VMEM)],
        out_specs=pl.BlockSpec(memory_space=pltpu.VMEM),
        scratch_shapes=[
            pltpu.VMEM((T_SHARD, D), jnp.float32),
            pltpu.VMEM((T_SHARD, D), jnp.float32),
            pltpu.SemaphoreType.DMA,
            pltpu.SemaphoreType.DMA,
        ],
        compiler_params=pltpu.CompilerParams(collective_id=1),
    )(partial)


def kernel(x, router, W1, W2):
    x_full, w_sel = _gather_gate(x, router)
    partial = _experts(x_full, w_sel, W1, W2)
    return _reduce_scatter(partial)
